# baseline (speedup 1.0000x reference)
import functools

import jax
import jax.numpy as jnp
from jax import lax
from jax.experimental import pallas as pl
from jax.experimental.pallas import tpu as pltpu
from jax.experimental.pallas import tpu_sc as plsc

H = 8
VOCAB = 100000
DIM = 64
B = 1024
T = 50
NTOK = B * T
NC = 2
NS = 16
NW = NC * NS
BPW = B // NW
TPW = BPW * T
NBH = BPW // 2
HCK = NBH * T
GP16 = HCK // 16

_mesh = plsc.VectorSubcoreMesh(core_axis_name="c", subcore_axis_name="s")


@functools.partial(
    pl.kernel,
    mesh=_mesh,
    compiler_params=pltpu.CompilerParams(use_tc_tiling_on_sc=False),
    out_type=[
        jax.ShapeDtypeStruct((H, B, T, DIM), jnp.float32),
        jax.ShapeDtypeStruct((NTOK,), jnp.int32),
    ],
    scratch_types=[
        pltpu.VMEM((BPW, T), jnp.int32),
        pltpu.VMEM((TPW,), jnp.int32),
        pltpu.VMEM((TPW,), jnp.int32),
        pltpu.VMEM((NBH, T, DIM), jnp.float32),
        pltpu.VMEM((NBH, T, DIM), jnp.float32),
        pltpu.SemaphoreType.DMA,
        pltpu.SemaphoreType.DMA,
        pltpu.SemaphoreType.DMA,
    ],
)
def _emb_kernel(codecs2_hbm, codecsf_hbm, w_hbm, emb_hbm, mask_hbm,
                idx2_v, idxf_v, macc_v, rows0_v, rows1_v,
                gsem, osem0, osem1):
    wid = lax.axis_index("s") * NC + lax.axis_index("c")
    tok0 = wid * TPW
    b0 = wid * BPW
    rows_bufs = (rows0_v, rows1_v)
    osems = (osem0, osem1)

    def init_mask(g, _):
        macc_v[pl.ds(g * 16, 16)] = jnp.full((16,), 1, jnp.int32)
        return 0

    lax.fori_loop(0, TPW // 16, init_mask, 0)

    def head(h, _):
        w_h = w_hbm.at[pl.ds(h * VOCAB, VOCAB)]
        pltpu.sync_copy(codecs2_hbm.at[pl.ds(h * B + b0, BPW)], idx2_v)
        pltpu.sync_copy(codecsf_hbm.at[pl.ds(h * NTOK + tok0, TPW)], idxf_v)

        for p in (0, 1):
            rows_v = rows_bufs[p]
            osem = osems[p]
            h0 = p * HCK

            def scan(g, orv, h0=h0):
                off = h0 + g * 16
                v = idxf_v[pl.ds(off, 16)]
                eq = jnp.where(v == 0, 1, 0).astype(jnp.int32)
                macc_v[pl.ds(off, 16)] = macc_v[pl.ds(off, 16)] & eq
                return orv | eq

            orv = lax.fori_loop(0, GP16, scan, jnp.zeros((16,), jnp.int32))
            any_zero = orv[0]
            for l in range(1, 16):
                any_zero = any_zero | orv[l]

            @pl.when(h > 0)
            def _drain_prev(rows_v=rows_v, osem=osem):
                pltpu.make_async_copy(
                    rows_v, emb_hbm.at[0, pl.ds(0, NBH)], osem).wait()

            descs = []
            for lb in range(NBH):
                descs.append(pltpu.async_copy(
                    w_h.at[idx2_v.at[p * NBH + lb]],
                    rows_v.at[lb], gsem))
            for d in descs:
                d.wait()

            @pl.when(any_zero > 0)
            def _fix(rows_v=rows_v, h0=h0):
                zeros = jnp.zeros((16,), jnp.float32)

                def fix_group(g, _):
                    v16 = idxf_v[pl.ds(h0 + g * 16, 16)]
                    for l in range(16):
                        @pl.when(v16[l] == 0)
                        def _z(l=l):
                            o = g * 16 + l
                            for q in range(DIM // 16):
                                rows_v[o // T, o % T, pl.ds(q * 16, 16)] = (
                                    zeros)
                    return 0

                lax.fori_loop(0, GP16, fix_group, 0)

            pltpu.async_copy(
                rows_v, emb_hbm.at[h, pl.ds(b0 + p * NBH, NBH)], osem)
        return 0

    lax.fori_loop(0, H, head, 0)

    for p in (0, 1):
        pltpu.make_async_copy(
            rows_bufs[p], emb_hbm.at[0, pl.ds(0, NBH)], osems[p]).wait()

    pltpu.sync_copy(macc_v, mask_hbm.at[pl.ds(tok0, TPW)])


def kernel(codecs, W):
    codecs_t = jnp.transpose(codecs.reshape(NTOK, H)).reshape(H * NTOK)
    codecs_2d = codecs_t.reshape(H * B, T)
    w_flat = W.reshape(H * VOCAB, DIM)
    emb, mask_i32 = _emb_kernel(codecs_2d, codecs_t, w_flat)
    mask = mask_i32.reshape(B, T).astype(bool)
    return (emb, mask)
